# baseline (device time: 18273 ns/iter reference)
import jax
import jax.numpy as jnp
from jax import lax
from jax.experimental import pallas as pl
from jax.experimental.pallas import tpu as pltpu

HALF = 512
D = 512
EPS = 1e-6


def kernel(partial, gamma):
    g2 = gamma.reshape(1, D)

    def body(x_ref, g_ref, out_ref, comm_ref, send_sem, recv_sem):
        my_x = lax.axis_index("x")
        my_y = lax.axis_index("y")
        nbr = (my_x, 1 - my_y)

        barrier = pltpu.get_barrier_semaphore()
        pl.semaphore_signal(
            barrier, inc=1, device_id=nbr, device_id_type=pl.DeviceIdType.MESH
        )
        pl.semaphore_wait(barrier, 1)

        rdma = pltpu.make_async_remote_copy(
            src_ref=x_ref.at[0, pl.ds((1 - my_y) * HALF, HALF), :],
            dst_ref=comm_ref,
            send_sem=send_sem,
            recv_sem=recv_sem,
            device_id=nbr,
            device_id_type=pl.DeviceIdType.MESH,
        )
        rdma.start()
        rdma.wait()

        acc = x_ref[0, pl.ds(my_y * HALF, HALF), :] + comm_ref[:, :]
        ms = jnp.mean(acc * acc, axis=-1, keepdims=True)
        out_ref[:, :] = acc * lax.rsqrt(ms + EPS) * g_ref[0, :]

    return pl.pallas_call(
        body,
        out_shape=jax.ShapeDtypeStruct((HALF, D), jnp.float32),
        in_specs=[
            pl.BlockSpec(memory_space=pltpu.VMEM),
            pl.BlockSpec(memory_space=pltpu.VMEM),
        ],
        out_specs=pl.BlockSpec(memory_space=pltpu.VMEM),
        scratch_shapes=[
            pltpu.VMEM((HALF, D), jnp.float32),
            pltpu.SemaphoreType.DMA,
            pltpu.SemaphoreType.DMA,
        ],
        compiler_params=pltpu.CompilerParams(collective_id=0),
    )(partial, g2)


# device time: 16531 ns/iter; 1.1054x vs baseline; 1.1054x over previous
import jax
import jax.numpy as jnp
from jax import lax
from jax.experimental import pallas as pl
from jax.experimental.pallas import tpu as pltpu

HALF = 512
D = 512
EPS = 1e-6
RQ = HALF // 2
C = 4
RC = RQ // C

_MESH = pl.DeviceIdType.MESH


def kernel(partial, gamma):
    g2 = gamma.reshape(1, D)

    def body(x_ref, g_ref, out_ref, comm_ref, ysend, yrecv, xsend, xrecv):
        my_x = lax.axis_index("x")
        my_y = lax.axis_index("y")
        ynbr = (my_x, 1 - my_y)
        xnbr = (1 - my_x, my_y)

        barrier = pltpu.get_barrier_semaphore()
        pl.semaphore_signal(barrier, inc=1, device_id=ynbr, device_id_type=_MESH)
        pl.semaphore_signal(barrier, inc=1, device_id=xnbr, device_id_type=_MESH)
        pl.semaphore_wait(barrier, 2)

        my_q = my_x * RQ
        send_base = (1 - my_y) * HALF + my_q
        acc_base = my_y * HALF + my_q

        y_rdmas = []
        for c in range(C):
            r = pltpu.make_async_remote_copy(
                src_ref=x_ref.at[0, pl.ds(send_base + c * RC, RC), :],
                dst_ref=comm_ref.at[pl.ds(c * RC, RC), :],
                send_sem=ysend.at[c],
                recv_sem=yrecv.at[c],
                device_id=ynbr,
                device_id_type=_MESH,
            )
            r.start()
            y_rdmas.append(r)

        x_rdmas = []
        for c in range(C):
            y_rdmas[c].wait_recv()
            acc = (
                x_ref[0, pl.ds(acc_base + c * RC, RC), :]
                + comm_ref[pl.ds(c * RC, RC), :]
            )
            ms = jnp.mean(acc * acc, axis=-1, keepdims=True)
            out_ref[pl.ds(my_q + c * RC, RC), :] = (
                acc * lax.rsqrt(ms + EPS) * g_ref[0, :]
            )
            r = pltpu.make_async_remote_copy(
                src_ref=out_ref.at[pl.ds(my_q + c * RC, RC), :],
                dst_ref=out_ref.at[pl.ds(my_q + c * RC, RC), :],
                send_sem=xsend.at[c],
                recv_sem=xrecv.at[c],
                device_id=xnbr,
                device_id_type=_MESH,
            )
            r.start()
            x_rdmas.append(r)

        for c in range(C):
            x_rdmas[c].wait_recv()
        for c in range(C):
            y_rdmas[c].wait_send()
            x_rdmas[c].wait_send()

    return pl.pallas_call(
        body,
        out_shape=jax.ShapeDtypeStruct((HALF, D), jnp.float32),
        in_specs=[
            pl.BlockSpec(memory_space=pltpu.VMEM),
            pl.BlockSpec(memory_space=pltpu.VMEM),
        ],
        out_specs=pl.BlockSpec(memory_space=pltpu.VMEM),
        scratch_shapes=[
            pltpu.VMEM((RQ, D), jnp.float32),
            pltpu.SemaphoreType.DMA((C,)),
            pltpu.SemaphoreType.DMA((C,)),
            pltpu.SemaphoreType.DMA((C,)),
            pltpu.SemaphoreType.DMA((C,)),
        ],
        compiler_params=pltpu.CompilerParams(collective_id=0),
    )(partial, g2)


# device time: 16066 ns/iter; 1.1374x vs baseline; 1.0289x over previous
import jax
import jax.numpy as jnp
from jax import lax
from jax.experimental import pallas as pl
from jax.experimental.pallas import tpu as pltpu

HALF = 512
D = 512
EPS = 1e-6
RQ = HALF // 2
C = 8
RC = RQ // C

_MESH = pl.DeviceIdType.MESH


def kernel(partial, gamma):
    g2 = gamma.reshape(1, D)

    def body(x_ref, g_ref, out_ref, acc_ref, snd_ref, comm_ref, res_ref,
             copy_sems, store_sems, ysend, yrecv, xsend, xrecv):
        my_x = lax.axis_index("x")
        my_y = lax.axis_index("y")
        ynbr = (my_x, 1 - my_y)
        xnbr = (1 - my_x, my_y)

        my_q = my_x * RQ
        send_base = (1 - my_y) * HALF + my_q
        acc_base = my_y * HALF + my_q

        stage_snd = pltpu.make_async_copy(
            x_ref.at[0, pl.ds(send_base, RQ), :], snd_ref, copy_sems.at[0]
        )
        stage_snd.start()
        stage_acc = pltpu.make_async_copy(
            x_ref.at[0, pl.ds(acc_base, RQ), :], acc_ref, copy_sems.at[1]
        )
        stage_acc.start()

        barrier = pltpu.get_barrier_semaphore()
        pl.semaphore_signal(barrier, inc=1, device_id=(my_x, my_y), device_id_type=_MESH)
        pl.semaphore_signal(barrier, inc=1, device_id=(my_x, my_y), device_id_type=_MESH)
        pl.semaphore_wait(barrier, 2)
        stage_snd.wait()

        y_rdmas = []
        for c in range(C):
            r = pltpu.make_async_remote_copy(
                src_ref=snd_ref.at[pl.ds(c * RC, RC), :],
                dst_ref=comm_ref.at[pl.ds(c * RC, RC), :],
                send_sem=ysend.at[c],
                recv_sem=yrecv.at[c],
                device_id=ynbr,
                device_id_type=_MESH,
            )
            r.start()
            y_rdmas.append(r)

        stage_acc.wait()

        x_rdmas = []
        stores = []
        for c in range(C):
            y_rdmas[c].wait_recv()
            acc = acc_ref[pl.ds(c * RC, RC), :] + comm_ref[pl.ds(c * RC, RC), :]
            ms = jnp.mean(acc * acc, axis=-1, keepdims=True)
            res_ref[pl.ds(c * RC, RC), :] = acc * lax.rsqrt(ms + EPS) * g_ref[0, :]
            st = pltpu.make_async_copy(
                res_ref.at[pl.ds(c * RC, RC), :],
                out_ref.at[pl.ds(my_q + c * RC, RC), :],
                store_sems.at[c],
            )
            st.start()
            stores.append(st)
            r = pltpu.make_async_remote_copy(
                src_ref=res_ref.at[pl.ds(c * RC, RC), :],
                dst_ref=out_ref.at[pl.ds(my_q + c * RC, RC), :],
                send_sem=xsend.at[c],
                recv_sem=xrecv.at[c],
                device_id=xnbr,
                device_id_type=_MESH,
            )
            r.start()
            x_rdmas.append(r)

        for c in range(C):
            x_rdmas[c].wait_recv()
        for c in range(C):
            stores[c].wait()
            y_rdmas[c].wait_send()
            x_rdmas[c].wait_send()

    return pl.pallas_call(
        body,
        out_shape=jax.ShapeDtypeStruct((HALF, D), jnp.float32),
        in_specs=[
            pl.BlockSpec(memory_space=pl.ANY),
            pl.BlockSpec(memory_space=pltpu.VMEM),
        ],
        out_specs=pl.BlockSpec(memory_space=pl.ANY),
        scratch_shapes=[
            pltpu.VMEM((RQ, D), jnp.float32),
            pltpu.VMEM((RQ, D), jnp.float32),
            pltpu.VMEM((RQ, D), jnp.float32),
            pltpu.VMEM((RQ, D), jnp.float32),
            pltpu.SemaphoreType.DMA((2,)),
            pltpu.SemaphoreType.DMA((C,)),
            pltpu.SemaphoreType.DMA((C,)),
            pltpu.SemaphoreType.DMA((C,)),
            pltpu.SemaphoreType.DMA((C,)),
            pltpu.SemaphoreType.DMA((C,)),
        ],
        compiler_params=pltpu.CompilerParams(collective_id=0),
    )(partial, g2)


# device time: 14336 ns/iter; 1.2746x vs baseline; 1.1207x over previous
import jax
import jax.numpy as jnp
from jax import lax
from jax.experimental import pallas as pl
from jax.experimental.pallas import tpu as pltpu

HALF = 512
D = 512
EPS = 1e-6
RQ = HALF // 2
DUP = 48

SCHED = (64, 48, 40, 32, 24, DUP, DUP)
NX = 5
TOT = RQ + DUP
assert sum(SCHED) == TOT
OFFS = [sum(SCHED[:i]) for i in range(len(SCHED))]

_MESH = pl.DeviceIdType.MESH


def kernel(partial, gamma):
    g2 = gamma.reshape(1, D)

    def body(x_ref, g_ref, out_ref, acc_ref, snd_ref, comm_ref, res_ref,
             gv_ref, copy_sems, store_sems, ysend, yrecv, xsend, xrecv):
        my_x = lax.axis_index("x")
        my_y = lax.axis_index("y")
        ynbr = (my_x, 1 - my_y)
        xnbr = (1 - my_x, my_y)

        my_q = my_x * RQ
        nbr_q = (1 - my_x) * RQ
        other = (1 - my_y) * HALF
        mine = my_y * HALF

        barrier = pltpu.get_barrier_semaphore()
        pl.semaphore_signal(barrier, inc=1, device_id=ynbr, device_id_type=_MESH)
        pl.semaphore_signal(barrier, inc=1, device_id=xnbr, device_id_type=_MESH)

        stages = [
            pltpu.make_async_copy(
                x_ref.at[0, pl.ds(other + my_q, RQ), :],
                snd_ref.at[pl.ds(0, RQ), :], copy_sems.at[0]),
            pltpu.make_async_copy(
                x_ref.at[0, pl.ds(other + nbr_q + RQ - DUP, DUP), :],
                snd_ref.at[pl.ds(RQ, DUP), :], copy_sems.at[1]),
            pltpu.make_async_copy(
                x_ref.at[0, pl.ds(mine + my_q, RQ), :],
                acc_ref.at[pl.ds(0, RQ), :], copy_sems.at[2]),
            pltpu.make_async_copy(
                x_ref.at[0, pl.ds(mine + nbr_q + RQ - DUP, DUP), :],
                acc_ref.at[pl.ds(RQ, DUP), :], copy_sems.at[3]),
            pltpu.make_async_copy(g_ref, gv_ref, copy_sems.at[4]),
        ]
        for s in stages:
            s.start()

        pl.semaphore_wait(barrier, 2)
        stages[0].wait()
        stages[1].wait()

        y_rdmas = []
        for c, (off, n) in enumerate(zip(OFFS, SCHED)):
            r = pltpu.make_async_remote_copy(
                src_ref=snd_ref.at[pl.ds(off, n), :],
                dst_ref=comm_ref.at[pl.ds(off, n), :],
                send_sem=ysend.at[c],
                recv_sem=yrecv.at[c],
                device_id=ynbr,
                device_id_type=_MESH,
            )
            r.start()
            y_rdmas.append(r)

        stages[2].wait()
        stages[3].wait()
        stages[4].wait()

        x_rdmas = []
        stores = []
        for c, (off, n) in enumerate(zip(OFFS, SCHED)):
            out_off = my_q + off if c < NX + 1 else nbr_q + RQ - DUP
            y_rdmas[c].wait_recv()
            acc = acc_ref[pl.ds(off, n), :] + comm_ref[pl.ds(off, n), :]
            ms = jnp.mean(acc * acc, axis=-1, keepdims=True)
            res_ref[pl.ds(off, n), :] = acc * lax.rsqrt(ms + EPS) * gv_ref[0, :]
            if c < NX:
                r = pltpu.make_async_remote_copy(
                    src_ref=res_ref.at[pl.ds(off, n), :],
                    dst_ref=out_ref.at[pl.ds(out_off, n), :],
                    send_sem=xsend.at[c],
                    recv_sem=xrecv.at[c],
                    device_id=xnbr,
                    device_id_type=_MESH,
                )
                r.start()
                x_rdmas.append(r)
            st = pltpu.make_async_copy(
                res_ref.at[pl.ds(off, n), :],
                out_ref.at[pl.ds(out_off, n), :],
                store_sems.at[c],
            )
            st.start()
            stores.append(st)

        for r in x_rdmas:
            r.wait_recv()
        for st in stores:
            st.wait()
        for r in y_rdmas:
            r.wait_send()
        for r in x_rdmas:
            r.wait_send()

    nc = len(SCHED)
    return pl.pallas_call(
        body,
        out_shape=jax.ShapeDtypeStruct((HALF, D), jnp.float32),
        in_specs=[
            pl.BlockSpec(memory_space=pltpu.MemorySpace.HBM),
            pl.BlockSpec(memory_space=pltpu.MemorySpace.HBM),
        ],
        out_specs=pl.BlockSpec(memory_space=pltpu.MemorySpace.HBM),
        scratch_shapes=[
            pltpu.VMEM((TOT, D), jnp.float32),
            pltpu.VMEM((TOT, D), jnp.float32),
            pltpu.VMEM((TOT, D), jnp.float32),
            pltpu.VMEM((TOT, D), jnp.float32),
            pltpu.VMEM((1, D), jnp.float32),
            pltpu.SemaphoreType.DMA((5,)),
            pltpu.SemaphoreType.DMA((nc,)),
            pltpu.SemaphoreType.DMA((nc,)),
            pltpu.SemaphoreType.DMA((nc,)),
            pltpu.SemaphoreType.DMA((NX,)),
            pltpu.SemaphoreType.DMA((NX,)),
        ],
        compiler_params=pltpu.CompilerParams(collective_id=0),
    )(
        pltpu.with_memory_space_constraint(partial, pltpu.MemorySpace.HBM),
        pltpu.with_memory_space_constraint(g2, pltpu.MemorySpace.HBM),
    )
